# baseline (speedup 1.0000x reference)
import math
import functools

import jax
import jax.numpy as jnp
from jax import lax
from jax.experimental import pallas as pl
from jax.experimental.pallas import tpu as pltpu
from jax.experimental.pallas import tpu_sc as plsc

K = 16
BIG = 1e30


def _ceil_to(n, m):
    return ((n + m - 1) // m) * m


def _pad_rows(a, n, val=0.0):
    if a.shape[0] == n:
        return a
    pad = [(0, n - a.shape[0])] + [(0, 0)] * (a.ndim - 1)
    return jnp.pad(a, pad, constant_values=val)


def _dot(a, b_t):
    return jax.lax.dot_general(a, b_t, (((1,), (1,)), ((), ())),
                               preferred_element_type=jnp.float32)


def _knn_body(q_ref, bT_ref, out_ref, *, n_valid, qb, self_ex):
    i = pl.program_id(0)
    q = q_ref[...]
    bT = bT_ref[...]
    cross = jax.lax.dot_general(q, bT, (((1,), (0,)), ((), ())),
                                preferred_element_type=jnp.float32)
    qn = jnp.sum(q * q, axis=1, keepdims=True)
    bn2 = jnp.sum(bT * bT, axis=0, keepdims=True)
    d = qn + bn2 - 2.0 * cross
    col = jax.lax.broadcasted_iota(jnp.int32, d.shape, 1)
    d = jnp.where(col >= n_valid, BIG, d)
    if self_ex:
        row = i * qb + jax.lax.broadcasted_iota(jnp.int32, d.shape, 0)
        d = jnp.where(row == col, d + 1e10, d)
    idxs = []
    for _ in range(K):
        aj = jnp.argmin(d, axis=1).astype(jnp.int32)[:, None]
        idxs.append(aj)
        d = jnp.where(col == aj, BIG, d)
    out_ref[...] = jnp.concatenate(idxs, axis=1)


def _knn_call(q, b, self_ex):
    nq, nb = q.shape[0], b.shape[0]
    qb = 256
    nqp = _ceil_to(nq, qb)
    nbp = _ceil_to(nb, 128)
    qp = _pad_rows(q, nqp)
    bT = jnp.transpose(_pad_rows(b, nbp, 1e6))
    body = functools.partial(_knn_body, n_valid=nb, qb=qb, self_ex=self_ex)
    out = pl.pallas_call(
        body,
        grid=(nqp // qb,),
        in_specs=[pl.BlockSpec((qb, 3), lambda i: (i, 0)),
                  pl.BlockSpec((3, nbp), lambda i: (0, 0))],
        out_specs=pl.BlockSpec((qb, K), lambda i: (i, 0)),
        out_shape=jax.ShapeDtypeStruct((nqp, K), jnp.int32),
    )(qp, bT)
    return out[:nq]


def _fps_body(posr_ref, posc_ref, out_ref, dists_ref, *, n_valid, m, lanes):
    xs = posc_ref[0]
    ys = posc_ref[1]
    zs = posc_ref[2]
    shp = (8, lanes)
    flat = (jax.lax.broadcasted_iota(jnp.int32, shp, 0) * lanes
            + jax.lax.broadcasted_iota(jnp.int32, shp, 1))
    p0 = posr_ref[0:1, :]
    d0 = (xs - p0[0, 0]) ** 2 + (ys - p0[0, 1]) ** 2 + (zs - p0[0, 2]) ** 2
    dists_ref[...] = jnp.where(flat < n_valid, d0, -1.0)
    out_ref[0] = jnp.int32(0)

    def it(i, carry):
        dd = dists_ref[...]
        gv = jnp.max(jnp.max(dd, axis=0, keepdims=True), axis=1, keepdims=True)
        wf = jnp.where(dd == gv, flat, jnp.int32(2 ** 30))
        nxt = jnp.min(jnp.min(wf, axis=0, keepdims=True),
                      axis=1, keepdims=True)[0, 0].astype(jnp.int32)
        out_ref[i] = nxt
        pr = posr_ref[pl.ds(nxt, 1), :]
        dn = (xs - pr[0, 0]) ** 2 + (ys - pr[0, 1]) ** 2 + (zs - pr[0, 2]) ** 2
        dists_ref[...] = jnp.minimum(dd, dn)
        return carry

    jax.lax.fori_loop(1, m, it, 0)


def _fps_call(pos, m):
    n = pos.shape[0]
    lanes = _ceil_to(-(-n // 8), 128)
    npad = 8 * lanes
    posr = _pad_rows(pos, npad, 1e6)
    posc = jnp.transpose(posr).reshape(3, 8, lanes)
    body = functools.partial(_fps_body, n_valid=n, m=m, lanes=lanes)
    return pl.pallas_call(
        body,
        out_specs=pl.BlockSpec(memory_space=pltpu.SMEM),
        out_shape=jax.ShapeDtypeStruct((m,), jnp.int32),
        scratch_shapes=[pltpu.VMEM((8, lanes), jnp.float32)],
    )(posr, posc)


_NW = 32
_NC = 2


def _sc_gather_call(tables, idx_flat):
    bsz = idx_flat.shape[0]
    ds = [t.shape[1] for t in tables]
    row_b = 4 * (sum(ds) + 1)
    bpw = -(-bsz // _NW)
    s_max = max(8, (400_000 // row_b) // 8 * 8)
    nc = -(-bpw // s_max)
    s = -(-bpw // (nc * 8)) * 8
    bpw = nc * s
    bpad = _NW * bpw
    idxp = jnp.pad(idx_flat, (0, bpad - bsz))
    nt = len(tables)
    out_t = [jax.ShapeDtypeStruct((bpad, d), jnp.float32) for d in ds]
    scratch = ([pltpu.VMEM((s,), jnp.int32)]
               + [pltpu.VMEM((s, d), jnp.float32) for d in ds]
               + [pltpu.SemaphoreType.DMA])

    @functools.partial(
        pl.kernel,
        mesh=plsc.VectorSubcoreMesh(core_axis_name="c", subcore_axis_name="s"),
        out_type=out_t, scratch_types=scratch)
    def k(*refs):
        tabs = refs[:nt]
        idx_hbm = refs[nt]
        outs = refs[nt + 1:nt + 1 + nt]
        idx_v = refs[nt + 1 + nt]
        rows = refs[nt + 2 + nt:nt + 2 + 2 * nt]
        sem = refs[-1]
        wid = lax.axis_index("s") * _NC + lax.axis_index("c")
        base = wid * bpw

        def chunk(j, c):
            off = base + j * s
            pltpu.sync_copy(idx_hbm.at[pl.ds(off, s)], idx_v)
            for t in range(nt):
                pltpu.async_copy(tabs[t].at[idx_v], rows[t], sem).wait()
                pltpu.sync_copy(rows[t], outs[t].at[pl.ds(off, s)])
            return c

        lax.fori_loop(0, nc, chunk, 0)

    res = k(*tables, idxp)
    return list(res) if isinstance(res, (list, tuple)) else [res]


def _pre_body(x_ref, wi_ref, bi_ref, wl_ref, ws_ref, wd_ref,
              va_ref, ad_ref, *, cout, slot):
    h = jnp.maximum(_dot(x_ref[...], wi_ref[...]) + bi_ref[...], 0.0)
    v = _dot(h, wl_ref[...])
    a_src = _dot(h, ws_ref[...])
    ad_ref[...] = _dot(h, wd_ref[...])
    if slot == cout:
        va_ref[...] = jnp.concatenate([a_src, v], axis=1)
    else:
        z = jnp.zeros((h.shape[0], slot - cout), jnp.float32)
        va_ref[...] = jnp.concatenate([a_src, z, v, z], axis=1)


def _pre_call(x, blk):
    n, cin = x.shape
    cout = blk['W_lin'].shape[0]
    slot = cout
    rb = 256
    npad = _ceil_to(n, rb)
    xp = _pad_rows(x, npad)
    bi = blk['lin_in_b'][None, :]
    full = lambda a: pl.BlockSpec(a.shape, lambda i: (0,) * a.ndim)
    body = functools.partial(_pre_body, cout=cout, slot=slot)
    outs = pl.pallas_call(
        body,
        grid=(npad // rb,),
        in_specs=[pl.BlockSpec((rb, cin), lambda i: (i, 0)),
                  full(blk['lin_in_W']), full(bi), full(blk['W_lin']),
                  full(blk['W_src']), full(blk['W_dst'])],
        out_specs=[pl.BlockSpec((rb, 2 * slot), lambda i: (i, 0)),
                   pl.BlockSpec((rb, cout), lambda i: (i, 0))],
        out_shape=[jax.ShapeDtypeStruct((npad, 2 * slot), jnp.float32),
                   jax.ShapeDtypeStruct((npad, cout), jnp.float32)],
    )(xp, blk['lin_in_W'], bi, blk['W_lin'], blk['W_src'], blk['W_dst'])
    return outs[0], outs[1]


def _edge_body(posq_ref, adst_ref, pn_ref, g_ref,
               pw1_ref, pb1_ref, pw2_ref, pb2_ref,
               aw1_ref, ab1_ref, aw2_ref, ab2_ref,
               ow_ref, ob_ref, out_ref, *, b, cout, slot):
    nk = K + 1
    rel = posq_ref[...][None] - pn_ref[:, :, 0:3]
    rel2 = rel.reshape(nk * b, 3)
    t = jnp.maximum(_dot(rel2, pw1_ref[...]) + pb1_ref[...], 0.0)
    delta2 = _dot(t, pw2_ref[...]) + pb2_ref[...]
    delta = delta2.reshape(nk, b, cout)
    asn = g_ref[:, :, 0:cout]
    vn = g_ref[:, :, slot:slot + cout]
    alpha = adst_ref[...][None] - asn + delta
    t2 = jnp.maximum(_dot(alpha.reshape(nk * b, cout), aw1_ref[...])
                     + ab1_ref[...], 0.0)
    alpha = (_dot(t2, aw2_ref[...]) + ab2_ref[...]).reshape(nk, b, cout)
    mx = jnp.max(alpha, axis=0, keepdims=True)
    e = jnp.exp(alpha - mx)
    a = e / jnp.sum(e, axis=0, keepdims=True)
    outv = jnp.sum(a * (vn + delta), axis=0)
    y = _dot(outv, ow_ref[...]) + ob_ref[...]
    out_ref[...] = jnp.maximum(y, 0.0)


def _edge_call(pos, adst, pnp, gnp, blk, b, npad):
    n = pos.shape[0]
    cout = blk['W_lin'].shape[0]
    slot = cout
    nk = K + 1
    posq = _pad_rows(pos, npad)
    adstp = _pad_rows(adst, npad)
    w = {k: blk[k] for k in ('pos_W1', 'pos_W2', 'att_W1', 'att_W2',
                             'lin_out_W')}
    bias = {k: blk[k][None, :] for k in ('pos_b1', 'pos_b2', 'att_b1',
                                         'att_b2', 'lin_out_b')}
    full = lambda a: pl.BlockSpec(a.shape, lambda i: (0,) * a.ndim)
    body = functools.partial(_edge_body, b=b, cout=cout, slot=slot)
    y = pl.pallas_call(
        body,
        grid=(npad // b,),
        in_specs=[pl.BlockSpec((b, 3), lambda i: (i, 0)),
                  pl.BlockSpec((b, cout), lambda i: (i, 0)),
                  pl.BlockSpec((nk, b, 3), lambda i: (0, i, 0)),
                  pl.BlockSpec((nk, b, 2 * slot), lambda i: (0, i, 0)),
                  full(w['pos_W1']), full(bias['pos_b1']),
                  full(w['pos_W2']), full(bias['pos_b2']),
                  full(w['att_W1']), full(bias['att_b1']),
                  full(w['att_W2']), full(bias['att_b2']),
                  full(w['lin_out_W']), full(bias['lin_out_b'])],
        out_specs=pl.BlockSpec((b, cout), lambda i: (i, 0)),
        out_shape=jax.ShapeDtypeStruct((npad, cout), jnp.float32),
    )(posq, adstp, pnp, gnp,
      w['pos_W1'], bias['pos_b1'], w['pos_W2'], bias['pos_b2'],
      w['att_W1'], bias['att_b1'], w['att_W2'], bias['att_b2'],
      w['lin_out_W'], bias['lin_out_b'])
    return y[:n]


def _bnlin_body(x_ref, w_ref, b_ref, g_ref, bt_ref, o_ref):
    y = _dot(x_ref[...], w_ref[...]) + b_ref[...]
    mu = jnp.mean(y, axis=0, keepdims=True)
    va = jnp.mean((y - mu) ** 2, axis=0, keepdims=True)
    o_ref[...] = jnp.maximum((y - mu) / jnp.sqrt(va + 1e-5) * g_ref[...]
                             + bt_ref[...], 0.0)


def _bnlin_call(x, w, b, g, beta):
    n = x.shape[0]
    cout = w.shape[0]
    return pl.pallas_call(
        _bnlin_body,
        out_shape=jax.ShapeDtypeStruct((n, cout), jnp.float32),
    )(x, w, b[None, :], g[None, :], beta[None, :])


def _downmax_body(xg_ref, o_ref):
    o_ref[...] = jnp.max(xg_ref[...], axis=0)


def _downmax_call(xg):
    _, m, c = xg.shape
    return pl.pallas_call(
        _downmax_body,
        out_shape=jax.ShapeDtypeStruct((m, c), jnp.float32),
    )(xg)


def _head_body(x_ref, w1_ref, b1_ref, w2_ref, b2_ref, w3_ref, b3_ref, o_ref):
    xm = jnp.mean(x_ref[...], axis=0, keepdims=True)
    h = jnp.maximum(_dot(xm, w1_ref[...]) + b1_ref[...], 0.0)
    h = jnp.maximum(_dot(h, w2_ref[...]) + b2_ref[...], 0.0)
    o = _dot(h, w3_ref[...]) + b3_ref[...]
    mx = jnp.max(o, axis=1, keepdims=True)
    lse = jnp.log(jnp.sum(jnp.exp(o - mx), axis=1, keepdims=True)) + mx
    o_ref[...] = o - lse


def _head_call(x, head):
    (w1, b1), (w2, b2), (w3, b3) = head
    return pl.pallas_call(
        _head_body,
        out_shape=jax.ShapeDtypeStruct((1, w3.shape[0]), jnp.float32),
    )(x, w1, b1[None, :], w2, b2[None, :], w3, b3[None, :])


def _block_call(x, pos, nbr, blk):
    n = x.shape[0]
    cout = blk['W_lin'].shape[0]
    b = 256 if cout <= 128 else (128 if cout == 256 else 64)
    npad = _ceil_to(n, b)
    va, adst = _pre_call(x, blk)
    nbrfT = jnp.concatenate(
        [jnp.arange(n, dtype=nbr.dtype)[None, :], jnp.transpose(nbr)], axis=0)
    nbrfT = jnp.pad(nbrfT, ((0, 0), (0, npad - n)))
    nk = K + 1
    slot2 = va.shape[1]
    pn = jnp.transpose(pos[jnp.transpose(nbrfT)], (1, 0, 2))
    gn = _sc_gather_call([va], nbrfT.reshape(-1))[0]
    gn = gn[:nk * npad].reshape(nk, npad, slot2)
    return _edge_call(pos, adst[:n], pn, gn, blk, b, npad)


def kernel(x, pos, batch, params):
    p = params
    x = _bnlin_call(x, p['in_W'], p['in_b'], p['in_g'], p['in_beta'])
    for i in range(len(p['blocks'])):
        blk = p['blocks'][i]
        down = p['downs'][i]
        nbr = _knn_call(pos, pos, self_ex=True)
        x = _block_call(x, pos, nbr, blk)
        n = pos.shape[0]
        m = int(math.ceil(n * 0.25))
        idc = _fps_call(pos, m)
        nbr2 = jnp.concatenate([idc[:, None], nbr[idc][:, :K - 1]], axis=1)
        x2 = _bnlin_call(x, down['W'], down['b'], down['g'], down['beta'])
        c = x2.shape[1]
        cslot = _ceil_to(c, 128)
        x2p = jnp.pad(x2, ((0, 0), (0, cslot - c))) if cslot != c else x2
        mp = _ceil_to(m, 16)
        nbr2T = jnp.pad(jnp.transpose(nbr2), ((0, 0), (0, mp - m)))
        xg = _sc_gather_call([x2p], nbr2T.reshape(-1))[0]
        xg = xg[:K * mp].reshape(K, mp, cslot)
        x = _downmax_call(xg)[:m, :c]
        pos = pos[idc]
    nbr = _knn_call(pos, pos, self_ex=True)
    x = _block_call(x, pos, nbr, p['final'])
    return _head_call(x, p['head'])


# speedup vs baseline: 1.2416x; 1.2416x over previous
import math
import functools

import jax
import jax.numpy as jnp
from jax import lax
from jax.experimental import pallas as pl
from jax.experimental.pallas import tpu as pltpu
from jax.experimental.pallas import tpu_sc as plsc

K = 16
BIG = 1e30


def _ceil_to(n, m):
    return ((n + m - 1) // m) * m


def _pad_rows(a, n, val=0.0):
    if a.shape[0] == n:
        return a
    pad = [(0, n - a.shape[0])] + [(0, 0)] * (a.ndim - 1)
    return jnp.pad(a, pad, constant_values=val)


def _dot(a, b_t):
    return jax.lax.dot_general(a, b_t, (((1,), (1,)), ((), ())),
                               preferred_element_type=jnp.float32)


def _knn_body(q_ref, bT_ref, out_ref, *, n_valid, qb, self_ex):
    i = pl.program_id(0)
    q = q_ref[...]
    bT = bT_ref[...]
    cross = jax.lax.dot_general(q, bT, (((1,), (0,)), ((), ())),
                                preferred_element_type=jnp.float32)
    qn = jnp.sum(q * q, axis=1, keepdims=True)
    bn2 = jnp.sum(bT * bT, axis=0, keepdims=True)
    d = qn + bn2 - 2.0 * cross
    col = jax.lax.broadcasted_iota(jnp.int32, d.shape, 1)
    d = jnp.where(col >= n_valid, BIG, d)
    if self_ex:
        row = i * qb + jax.lax.broadcasted_iota(jnp.int32, d.shape, 0)
        d = jnp.where(row == col, d + 1e10, d)
    idxs = []
    for _ in range(K):
        aj = jnp.argmin(d, axis=1).astype(jnp.int32)[:, None]
        idxs.append(aj)
        d = jnp.where(col == aj, BIG, d)
    out_ref[...] = jnp.concatenate(idxs, axis=1)


def _knn_call(q, b, self_ex):
    nq, nb = q.shape[0], b.shape[0]
    qb = 256
    nqp = _ceil_to(nq, qb)
    nbp = _ceil_to(nb, 128)
    qp = _pad_rows(q, nqp)
    bT = jnp.transpose(_pad_rows(b, nbp, 1e6))
    body = functools.partial(_knn_body, n_valid=nb, qb=qb, self_ex=self_ex)
    out = pl.pallas_call(
        body,
        grid=(nqp // qb,),
        in_specs=[pl.BlockSpec((qb, 3), lambda i: (i, 0)),
                  pl.BlockSpec((3, nbp), lambda i: (0, 0))],
        out_specs=pl.BlockSpec((qb, K), lambda i: (i, 0)),
        out_shape=jax.ShapeDtypeStruct((nqp, K), jnp.int32),
    )(qp, bT)
    return out[:nq]


def _fps_body(posr_ref, posT_ref, out_ref, dists_ref, *, n_valid, m):
    xs = posT_ref[0:1, :]
    ys = posT_ref[1:2, :]
    zs = posT_ref[2:3, :]
    col = jax.lax.broadcasted_iota(jnp.int32, xs.shape, 1)
    p0 = posr_ref[0:1, :]
    d0 = (xs - p0[0, 0]) ** 2 + (ys - p0[0, 1]) ** 2 + (zs - p0[0, 2]) ** 2
    dists_ref[...] = jnp.where(col < n_valid, d0, -1.0)
    out_ref[0] = jnp.int32(0)

    def it(i, carry):
        dd = dists_ref[...]
        nxt = jnp.argmax(dd, axis=1)[0].astype(jnp.int32)
        out_ref[i] = nxt
        pr = posr_ref[pl.ds(nxt, 1), :]
        dn = (xs - pr[0, 0]) ** 2 + (ys - pr[0, 1]) ** 2 + (zs - pr[0, 2]) ** 2
        dists_ref[...] = jnp.minimum(dd, dn)
        return carry

    jax.lax.fori_loop(1, m, it, 0)


def _fps_call(pos, m):
    n = pos.shape[0]
    npad = _ceil_to(n, 128)
    posr = _pad_rows(pos, npad, 1e6)
    posT = jnp.transpose(posr)
    body = functools.partial(_fps_body, n_valid=n, m=m)
    return pl.pallas_call(
        body,
        out_specs=pl.BlockSpec(memory_space=pltpu.SMEM),
        out_shape=jax.ShapeDtypeStruct((m,), jnp.int32),
        scratch_shapes=[pltpu.VMEM((1, npad), jnp.float32)],
    )(posr, posT)


_NW = 32
_NC = 2


def _sc_gather_call(tables, idx_flat):
    bsz = idx_flat.shape[0]
    ds = [t.shape[1] for t in tables]
    row_b = 4 * (sum(ds) + 1)
    bpw = -(-bsz // _NW)
    s_max = max(8, (400_000 // row_b) // 8 * 8)
    nc = -(-bpw // s_max)
    s = -(-bpw // (nc * 8)) * 8
    bpw = nc * s
    bpad = _NW * bpw
    idxp = jnp.pad(idx_flat, (0, bpad - bsz))
    nt = len(tables)
    out_t = [jax.ShapeDtypeStruct((bpad, d), jnp.float32) for d in ds]
    scratch = ([pltpu.VMEM((s,), jnp.int32)]
               + [pltpu.VMEM((s, d), jnp.float32) for d in ds]
               + [pltpu.SemaphoreType.DMA])

    @functools.partial(
        pl.kernel,
        mesh=plsc.VectorSubcoreMesh(core_axis_name="c", subcore_axis_name="s"),
        out_type=out_t, scratch_types=scratch)
    def k(*refs):
        tabs = refs[:nt]
        idx_hbm = refs[nt]
        outs = refs[nt + 1:nt + 1 + nt]
        idx_v = refs[nt + 1 + nt]
        rows = refs[nt + 2 + nt:nt + 2 + 2 * nt]
        sem = refs[-1]
        wid = lax.axis_index("s") * _NC + lax.axis_index("c")
        base = wid * bpw

        def chunk(j, c):
            off = base + j * s
            pltpu.sync_copy(idx_hbm.at[pl.ds(off, s)], idx_v)
            for t in range(nt):
                pltpu.async_copy(tabs[t].at[idx_v], rows[t], sem).wait()
                pltpu.sync_copy(rows[t], outs[t].at[pl.ds(off, s)])
            return c

        lax.fori_loop(0, nc, chunk, 0)

    res = k(*tables, idxp)
    return list(res) if isinstance(res, (list, tuple)) else [res]


def _pre_body(x_ref, pos_ref, wi_ref, bi_ref, wl_ref, ws_ref, wd_ref,
              va_ref, ad_ref, *, rb):
    h = jnp.maximum(_dot(x_ref[...], wi_ref[...]) + bi_ref[...], 0.0)
    v = _dot(h, wl_ref[...])
    a_src = _dot(h, ws_ref[...])
    ad_ref[...] = _dot(h, wd_ref[...])
    z = jnp.zeros((rb, 125), jnp.float32)
    va_ref[...] = jnp.concatenate([a_src, v, pos_ref[...], z], axis=1)


def _pre_call(x, pos, blk):
    n, cin = x.shape
    cout = blk['W_lin'].shape[0]
    rb = 256
    npad = _ceil_to(n, rb)
    xp = _pad_rows(x, npad)
    posp = _pad_rows(pos, npad)
    bi = blk['lin_in_b'][None, :]
    full = lambda a: pl.BlockSpec(a.shape, lambda i: (0,) * a.ndim)
    body = functools.partial(_pre_body, rb=rb)
    outs = pl.pallas_call(
        body,
        grid=(npad // rb,),
        in_specs=[pl.BlockSpec((rb, cin), lambda i: (i, 0)),
                  pl.BlockSpec((rb, 3), lambda i: (i, 0)),
                  full(blk['lin_in_W']), full(bi), full(blk['W_lin']),
                  full(blk['W_src']), full(blk['W_dst'])],
        out_specs=[pl.BlockSpec((rb, 2 * cout + 128), lambda i: (i, 0)),
                   pl.BlockSpec((rb, cout), lambda i: (i, 0))],
        out_shape=[jax.ShapeDtypeStruct((npad, 2 * cout + 128), jnp.float32),
                   jax.ShapeDtypeStruct((npad, cout), jnp.float32)],
    )(xp, posp, blk['lin_in_W'], bi, blk['W_lin'], blk['W_src'],
      blk['W_dst'])
    return outs[0], outs[1]


def _edge_body(posq_ref, adst_ref, g_ref,
               pw1_ref, pb1_ref, pw2_ref, pb2_ref,
               aw1_ref, ab1_ref, aw2_ref, ab2_ref,
               ow_ref, ob_ref, out_ref, *, b, cout, slot):
    nk = K + 1
    pn = g_ref[:, :, 2 * cout:2 * cout + 3]
    rel = posq_ref[...][None] - pn
    rel2 = rel.reshape(nk * b, 3)
    t = jnp.maximum(_dot(rel2, pw1_ref[...]) + pb1_ref[...], 0.0)
    delta2 = _dot(t, pw2_ref[...]) + pb2_ref[...]
    delta = delta2.reshape(nk, b, cout)
    asn = g_ref[:, :, 0:cout]
    vn = g_ref[:, :, cout:2 * cout]
    alpha = adst_ref[...][None] - asn + delta
    t2 = jnp.maximum(_dot(alpha.reshape(nk * b, cout), aw1_ref[...])
                     + ab1_ref[...], 0.0)
    alpha = (_dot(t2, aw2_ref[...]) + ab2_ref[...]).reshape(nk, b, cout)
    mx = jnp.max(alpha, axis=0, keepdims=True)
    e = jnp.exp(alpha - mx)
    a = e / jnp.sum(e, axis=0, keepdims=True)
    outv = jnp.sum(a * (vn + delta), axis=0)
    y = _dot(outv, ow_ref[...]) + ob_ref[...]
    out_ref[...] = jnp.maximum(y, 0.0)


def _edge_call(pos, adst, gnp, blk, b, npad):
    n = pos.shape[0]
    cout = blk['W_lin'].shape[0]
    slot = cout
    gw = 2 * cout + 128
    nk = K + 1
    posq = _pad_rows(pos, npad)
    adstp = _pad_rows(adst, npad)
    w = {k: blk[k] for k in ('pos_W1', 'pos_W2', 'att_W1', 'att_W2',
                             'lin_out_W')}
    bias = {k: blk[k][None, :] for k in ('pos_b1', 'pos_b2', 'att_b1',
                                         'att_b2', 'lin_out_b')}
    full = lambda a: pl.BlockSpec(a.shape, lambda i: (0,) * a.ndim)
    body = functools.partial(_edge_body, b=b, cout=cout, slot=slot)
    y = pl.pallas_call(
        body,
        grid=(npad // b,),
        in_specs=[pl.BlockSpec((b, 3), lambda i: (i, 0)),
                  pl.BlockSpec((b, cout), lambda i: (i, 0)),
                  pl.BlockSpec((nk, b, gw), lambda i: (0, i, 0)),
                  full(w['pos_W1']), full(bias['pos_b1']),
                  full(w['pos_W2']), full(bias['pos_b2']),
                  full(w['att_W1']), full(bias['att_b1']),
                  full(w['att_W2']), full(bias['att_b2']),
                  full(w['lin_out_W']), full(bias['lin_out_b'])],
        out_specs=pl.BlockSpec((b, cout), lambda i: (i, 0)),
        out_shape=jax.ShapeDtypeStruct((npad, cout), jnp.float32),
    )(posq, adstp, gnp,
      w['pos_W1'], bias['pos_b1'], w['pos_W2'], bias['pos_b2'],
      w['att_W1'], bias['att_b1'], w['att_W2'], bias['att_b2'],
      w['lin_out_W'], bias['lin_out_b'])
    return y[:n]


def _bnlin_body(x_ref, w_ref, b_ref, g_ref, bt_ref, o_ref):
    y = _dot(x_ref[...], w_ref[...]) + b_ref[...]
    mu = jnp.mean(y, axis=0, keepdims=True)
    va = jnp.mean((y - mu) ** 2, axis=0, keepdims=True)
    o_ref[...] = jnp.maximum((y - mu) / jnp.sqrt(va + 1e-5) * g_ref[...]
                             + bt_ref[...], 0.0)


def _bnlin_call(x, w, b, g, beta):
    n = x.shape[0]
    cout = w.shape[0]
    return pl.pallas_call(
        _bnlin_body,
        out_shape=jax.ShapeDtypeStruct((n, cout), jnp.float32),
    )(x, w, b[None, :], g[None, :], beta[None, :])


def _downmax_body(xg_ref, o_ref):
    o_ref[...] = jnp.max(xg_ref[...], axis=0)


def _downmax_call(xg):
    _, m, c = xg.shape
    return pl.pallas_call(
        _downmax_body,
        out_shape=jax.ShapeDtypeStruct((m, c), jnp.float32),
    )(xg)


def _head_body(x_ref, w1_ref, b1_ref, w2_ref, b2_ref, w3_ref, b3_ref, o_ref):
    xm = jnp.mean(x_ref[...], axis=0, keepdims=True)
    h = jnp.maximum(_dot(xm, w1_ref[...]) + b1_ref[...], 0.0)
    h = jnp.maximum(_dot(h, w2_ref[...]) + b2_ref[...], 0.0)
    o = _dot(h, w3_ref[...]) + b3_ref[...]
    mx = jnp.max(o, axis=1, keepdims=True)
    lse = jnp.log(jnp.sum(jnp.exp(o - mx), axis=1, keepdims=True)) + mx
    o_ref[...] = o - lse


def _head_call(x, head):
    (w1, b1), (w2, b2), (w3, b3) = head
    return pl.pallas_call(
        _head_body,
        out_shape=jax.ShapeDtypeStruct((1, w3.shape[0]), jnp.float32),
    )(x, w1, b1[None, :], w2, b2[None, :], w3, b3[None, :])


def _block_call(x, pos, nbr, blk):
    n = x.shape[0]
    cout = blk['W_lin'].shape[0]
    b = 256 if cout <= 128 else (128 if cout == 256 else 64)
    npad = _ceil_to(n, b)
    va, adst = _pre_call(x, pos, blk)
    nbrfT = jnp.concatenate(
        [jnp.arange(n, dtype=nbr.dtype)[None, :], jnp.transpose(nbr)], axis=0)
    nbrfT = jnp.pad(nbrfT, ((0, 0), (0, npad - n)))
    nk = K + 1
    gw = va.shape[1]
    gn = _sc_gather_call([va], nbrfT.reshape(-1))[0]
    gn = gn[:nk * npad].reshape(nk, npad, gw)
    return _edge_call(pos, adst[:n], gn, blk, b, npad)


def kernel(x, pos, batch, params):
    p = params
    x = _bnlin_call(x, p['in_W'], p['in_b'], p['in_g'], p['in_beta'])
    for i in range(len(p['blocks'])):
        blk = p['blocks'][i]
        down = p['downs'][i]
        nbr = _knn_call(pos, pos, self_ex=True)
        x = _block_call(x, pos, nbr, blk)
        n = pos.shape[0]
        m = int(math.ceil(n * 0.25))
        idc = _fps_call(pos, m)
        nbr2 = jnp.concatenate([idc[:, None], nbr[idc][:, :K - 1]], axis=1)
        x2 = _bnlin_call(x, down['W'], down['b'], down['g'], down['beta'])
        c = x2.shape[1]
        cslot = _ceil_to(c, 128)
        x2p = jnp.pad(x2, ((0, 0), (0, cslot - c))) if cslot != c else x2
        mp = _ceil_to(m, 16)
        nbr2T = jnp.pad(jnp.transpose(nbr2), ((0, 0), (0, mp - m)))
        xg = _sc_gather_call([x2p], nbr2T.reshape(-1))[0]
        xg = xg[:K * mp].reshape(K, mp, cslot)
        x = _downmax_call(xg)[:m, :c]
        pos = pos[idc]
    nbr = _knn_call(pos, pos, self_ex=True)
    x = _block_call(x, pos, nbr, p['final'])
    return _head_call(x, p['head'])
